# baseline (speedup 1.0000x reference)
import functools

import jax
import jax.numpy as jnp
from jax import lax
from jax.experimental import pallas as pl
from jax.experimental.pallas import tpu as pltpu
from jax.experimental.pallas import tpu_sc as plsc

_EPS = 1e-10
_CUTPER = 0.2

_NC = 2
_NS = 16
_NW = _NC * _NS
_R = 128


def _reduce_block(blk, idxv, W):
    R, H, _ = blk.shape
    w_t = idxv % W
    h_t = idxv // W
    lane = jax.lax.broadcasted_iota(jnp.int32, (R, 1, W), 2)
    mw = lane == w_t[:, None, None]
    wsum = jnp.sum(blk, axis=2)
    tv = jnp.sum(jnp.where(mw, blk, 0.0), axis=2)
    hh = jax.lax.broadcasted_iota(jnp.int32, (R, H), 1)
    mh = hh == h_t[:, None]
    return jnp.sum(wsum, axis=1), jnp.sum(jnp.where(mh, tv, 0.0), axis=1)


def _rows_dma_body(idx_ref, rm_any, sum_ref, val_ref,
                   *scratch, W, S, R3, NBUF, NG):
    g = pl.program_id(0)
    bufs = scratch[:NBUF]
    sems = scratch[NBUF:]

    def issue(bid, t):
        pltpu.make_async_copy(rm_any.at[pl.ds(bid * R3, R3)],
                              bufs[t], sems[t]).start()

    @pl.when(g == 0)
    def _prologue():
        for t in range(NBUF):
            issue(t, t)

    for t in range(NBUF):
        pltpu.make_async_copy(rm_any.at[pl.ds(0, R3)],
                              bufs[t], sems[t]).wait()
        idxv = idx_ref[0, 0, pl.ds(t * R3, R3)]
        ssum, sval = _reduce_block(bufs[t][...], idxv, W)
        sum_ref[0, 0, pl.ds(t * R3, R3)] = ssum
        val_ref[0, 0, pl.ds(t * R3, R3)] = sval

        @pl.when(g + 1 < NG)
        def _refill():
            issue((g + 1) * NBUF + t, t)


def _sc_gather_body(bnd_ref, idx_ref, out_ref, idx_v, gidx_v, out_v, sem,
                    *, S, HW, PW):
    wid = lax.axis_index("s") * _NC + lax.axis_index("c")
    base = wid * PW
    pltpu.sync_copy(idx_ref.at[pl.ds(base, PW)], idx_v)
    off = (base // S) * HW
    for j in range(PW // 16):
        gidx_v[pl.ds(j * 16, 16)] = idx_v[pl.ds(j * 16, 16)] + off
    pltpu.async_copy(bnd_ref.at[gidx_v], out_v, sem).wait()
    pltpu.sync_copy(out_v, out_ref.at[pl.ds(base, PW)])


def _loss_body(sr_ref, vr_ref, br_ref, sc_ref, vc_ref, bc_ref, out_ref, *, k):
    x_row = br_ref[...] * -jnp.log(_EPS + vr_ref[...] / sr_ref[...])
    x_col = bc_ref[...] * -jnp.log(_EPS + vc_ref[...] / sc_ref[...])
    le = (x_row <= x_col).astype(jnp.float32)
    cnt = jnp.sum(le, axis=1, keepdims=True)
    elig = cnt >= jnp.float32(k + 1)
    thr = jnp.min(jnp.where(elig, x_col, jnp.inf), keepdims=True)
    m = x_row > thr
    numer = jnp.sum(jnp.where(m, x_row, 0.0), axis=1, keepdims=True)
    denom = 1.0 + jnp.sum(jnp.where(m, br_ref[...], 0.0), axis=1, keepdims=True)
    out_ref[...] = numer / denom


def kernel(response_map, source_feature_1d_locations, boundaries):
    B, S, H, W = response_map.shape
    HW = H * W
    N = B * S
    R = _R
    NSB = S // R
    PW = N // _NW

    idx_i32 = source_feature_1d_locations.astype(jnp.int32)

    NBUF = 8
    R3 = 32
    GROUP = NBUF * R3
    NG = N // GROUP
    idx3 = idx_i32.reshape(NG, 1, GROUP)

    sums, vals = pl.pallas_call(
        functools.partial(_rows_dma_body, W=W, S=S, R3=R3, NBUF=NBUF, NG=NG),
        grid=(NG,),
        in_specs=[
            pl.BlockSpec((1, 1, GROUP), lambda g: (g, 0, 0)),
            pl.BlockSpec(memory_space=pl.ANY),
        ],
        out_specs=[
            pl.BlockSpec((1, 1, GROUP), lambda g: (g, 0, 0)),
            pl.BlockSpec((1, 1, GROUP), lambda g: (g, 0, 0)),
        ],
        out_shape=[
            jax.ShapeDtypeStruct((NG, 1, GROUP), jnp.float32),
            jax.ShapeDtypeStruct((NG, 1, GROUP), jnp.float32),
        ],
        scratch_shapes=(
            [pltpu.VMEM((R3, H, W), jnp.float32) for _ in range(NBUF)]
            + [pltpu.SemaphoreType.DMA for _ in range(NBUF)]),
        compiler_params=pltpu.CompilerParams(
            vmem_limit_bytes=100 * 1024 * 1024),
    )(idx3, response_map.reshape(N, H, W))

    sb_flat = pl.kernel(
        functools.partial(_sc_gather_body, S=S, HW=HW, PW=PW),
        out_type=jax.ShapeDtypeStruct((N,), jnp.float32),
        mesh=plsc.VectorSubcoreMesh(core_axis_name="c", subcore_axis_name="s",
                                    num_cores=_NC, num_subcores=_NS),
        scratch_types=[
            pltpu.VMEM((PW,), jnp.int32),
            pltpu.VMEM((PW,), jnp.int32),
            pltpu.VMEM((PW,), jnp.float32),
            pltpu.SemaphoreType.DMA,
        ],
    )(boundaries.reshape(B * HW), idx_i32.reshape(N))

    k = int(N * _CUTPER)
    sr = sums.reshape(1, N)
    vr = vals.reshape(1, N)
    br = sb_flat.reshape(1, N)

    loss = pl.pallas_call(
        functools.partial(_loss_body, k=k),
        out_shape=jax.ShapeDtypeStruct((1, 1), jnp.float32),
    )(sr, vr, br, sr.reshape(N, 1), vr.reshape(N, 1), br.reshape(N, 1))

    return loss.reshape(())


# speedup vs baseline: 1.0160x; 1.0160x over previous
import functools

import jax
import jax.numpy as jnp
from jax import lax
from jax.experimental import pallas as pl
from jax.experimental.pallas import tpu as pltpu
from jax.experimental.pallas import tpu_sc as plsc

_EPS = 1e-10
_CUTPER = 0.2

_NC = 2
_NS = 16
_NW = _NC * _NS
_R = 128


def _reduce_block(blk, idxv, W):
    R, H, _ = blk.shape
    w_t = idxv % W
    h_t = idxv // W
    lane = jax.lax.broadcasted_iota(jnp.int32, (R, 1, W), 2)
    mw = lane == w_t[:, None, None]
    wsum = jnp.sum(blk, axis=2)
    tv = jnp.sum(jnp.where(mw, blk, 0.0), axis=2)
    hh = jax.lax.broadcasted_iota(jnp.int32, (R, H), 1)
    mh = hh == h_t[:, None]
    return jnp.sum(wsum, axis=1), jnp.sum(jnp.where(mh, tv, 0.0), axis=1)


def _rows_dma_body(idx_ref, rm_any, sum_ref, val_ref,
                   *scratch, W, S, R3, NBUF, NG):
    g = pl.program_id(0)
    bufs = scratch[:NBUF]
    sems = scratch[NBUF:]

    def issue(bid, t):
        hh = R3 // 2
        pltpu.make_async_copy(rm_any.at[pl.ds(bid * R3, hh)],
                              bufs[t].at[pl.ds(0, hh)], sems[t]).start()
        pltpu.make_async_copy(rm_any.at[pl.ds(bid * R3 + hh, hh)],
                              bufs[t].at[pl.ds(hh, hh)], sems[t]).start()

    @pl.when(g == 0)
    def _prologue():
        for t in range(NBUF):
            issue(t, t)

    for t in range(NBUF):
        pltpu.make_async_copy(rm_any.at[pl.ds(0, R3)],
                              bufs[t], sems[t]).wait()
        idxv = idx_ref[0, 0, pl.ds(t * R3, R3)]
        ssum, sval = _reduce_block(bufs[t][...], idxv, W)
        sum_ref[0, 0, pl.ds(t * R3, R3)] = ssum
        val_ref[0, 0, pl.ds(t * R3, R3)] = sval

        @pl.when(g + 1 < NG)
        def _refill():
            issue((g + 1) * NBUF + t, t)


def _sc_gather_body(bnd_ref, idx_ref, out_ref, idx_v, gidx_v, out_v, sem,
                    *, S, HW, PW):
    wid = lax.axis_index("s") * _NC + lax.axis_index("c")
    base = wid * PW
    pltpu.sync_copy(idx_ref.at[pl.ds(base, PW)], idx_v)
    off = (base // S) * HW
    for j in range(PW // 16):
        gidx_v[pl.ds(j * 16, 16)] = idx_v[pl.ds(j * 16, 16)] + off
    pltpu.async_copy(bnd_ref.at[gidx_v], out_v, sem).wait()
    pltpu.sync_copy(out_v, out_ref.at[pl.ds(base, PW)])


def _loss_body(sr_ref, vr_ref, br_ref, sc_ref, vc_ref, bc_ref, out_ref, *, k):
    x_row = br_ref[...] * -jnp.log(_EPS + vr_ref[...] / sr_ref[...])
    x_col = bc_ref[...] * -jnp.log(_EPS + vc_ref[...] / sc_ref[...])
    le = (x_row <= x_col).astype(jnp.float32)
    cnt = jnp.sum(le, axis=1, keepdims=True)
    elig = cnt >= jnp.float32(k + 1)
    thr = jnp.min(jnp.where(elig, x_col, jnp.inf), keepdims=True)
    m = x_row > thr
    numer = jnp.sum(jnp.where(m, x_row, 0.0), axis=1, keepdims=True)
    denom = 1.0 + jnp.sum(jnp.where(m, br_ref[...], 0.0), axis=1, keepdims=True)
    out_ref[...] = numer / denom


def kernel(response_map, source_feature_1d_locations, boundaries):
    B, S, H, W = response_map.shape
    HW = H * W
    N = B * S
    R = _R
    NSB = S // R
    PW = N // _NW

    idx_i32 = source_feature_1d_locations.astype(jnp.int32)

    NBUF = 4
    R3 = 32
    GROUP = NBUF * R3
    NG = N // GROUP
    idx3 = idx_i32.reshape(NG, 1, GROUP)

    sums, vals = pl.pallas_call(
        functools.partial(_rows_dma_body, W=W, S=S, R3=R3, NBUF=NBUF, NG=NG),
        grid=(NG,),
        in_specs=[
            pl.BlockSpec((1, 1, GROUP), lambda g: (g, 0, 0)),
            pl.BlockSpec(memory_space=pl.ANY),
        ],
        out_specs=[
            pl.BlockSpec((1, 1, GROUP), lambda g: (g, 0, 0)),
            pl.BlockSpec((1, 1, GROUP), lambda g: (g, 0, 0)),
        ],
        out_shape=[
            jax.ShapeDtypeStruct((NG, 1, GROUP), jnp.float32),
            jax.ShapeDtypeStruct((NG, 1, GROUP), jnp.float32),
        ],
        scratch_shapes=(
            [pltpu.VMEM((R3, H, W), jnp.float32) for _ in range(NBUF)]
            + [pltpu.SemaphoreType.DMA for _ in range(NBUF)]),
        compiler_params=pltpu.CompilerParams(
            vmem_limit_bytes=100 * 1024 * 1024),
    )(idx3, response_map.reshape(N, H, W))

    sb_flat = pl.kernel(
        functools.partial(_sc_gather_body, S=S, HW=HW, PW=PW),
        out_type=jax.ShapeDtypeStruct((N,), jnp.float32),
        mesh=plsc.VectorSubcoreMesh(core_axis_name="c", subcore_axis_name="s",
                                    num_cores=_NC, num_subcores=_NS),
        scratch_types=[
            pltpu.VMEM((PW,), jnp.int32),
            pltpu.VMEM((PW,), jnp.int32),
            pltpu.VMEM((PW,), jnp.float32),
            pltpu.SemaphoreType.DMA,
        ],
    )(boundaries.reshape(B * HW), idx_i32.reshape(N))

    k = int(N * _CUTPER)
    sr = sums.reshape(1, N)
    vr = vals.reshape(1, N)
    br = sb_flat.reshape(1, N)

    loss = pl.pallas_call(
        functools.partial(_loss_body, k=k),
        out_shape=jax.ShapeDtypeStruct((1, 1), jnp.float32),
    )(sr, vr, br, sr.reshape(N, 1), vr.reshape(N, 1), br.reshape(N, 1))

    return loss.reshape(())


# speedup vs baseline: 1.0227x; 1.0066x over previous
import functools

import jax
import jax.numpy as jnp
from jax import lax
from jax.experimental import pallas as pl
from jax.experimental.pallas import tpu as pltpu
from jax.experimental.pallas import tpu_sc as plsc

_EPS = 1e-10
_CUTPER = 0.2

_NC = 2
_NS = 16
_NW = _NC * _NS
_R = 128


def _reduce_block(blk, idxv, W):
    R, H, _ = blk.shape
    w_t = idxv % W
    h_t = idxv // W
    lane = jax.lax.broadcasted_iota(jnp.int32, (R, 1, W), 2)
    mw = lane == w_t[:, None, None]
    wsum = jnp.sum(blk, axis=2)
    tv = jnp.sum(jnp.where(mw, blk, 0.0), axis=2)
    hh = jax.lax.broadcasted_iota(jnp.int32, (R, H), 1)
    mh = hh == h_t[:, None]
    return jnp.sum(wsum, axis=1), jnp.sum(jnp.where(mh, tv, 0.0), axis=1)


def _rows_dma_body(idx_ref, rm_any, sum_ref, val_ref,
                   *scratch, W, S, R3, NBUF, NG):
    g = pl.program_id(0)
    bufs = scratch[:NBUF]
    sems = scratch[NBUF:]

    def issue(bid, t):
        pltpu.make_async_copy(rm_any.at[pl.ds(bid * R3, R3)],
                              bufs[t], sems[t]).start()

    @pl.when(g == 0)
    def _prologue():
        for t in range(NBUF):
            issue(t, t)

    for t in range(NBUF):
        pltpu.make_async_copy(rm_any.at[pl.ds(0, R3)],
                              bufs[t], sems[t]).wait()
        idxv = idx_ref[0, 0, pl.ds(t * R3, R3)]
        ssum, sval = _reduce_block(bufs[t][...], idxv, W)
        sum_ref[0, 0, pl.ds(t * R3, R3)] = ssum
        val_ref[0, 0, pl.ds(t * R3, R3)] = sval

        @pl.when(g + 1 < NG)
        def _refill():
            issue((g + 1) * NBUF + t, t)


def _sc_gather_body(bnd_ref, idx_ref, out_ref, idx_v, gidx_v, out_v, sem,
                    *, S, HW, PW):
    wid = lax.axis_index("s") * _NC + lax.axis_index("c")
    base = wid * PW
    pltpu.sync_copy(idx_ref.at[pl.ds(base, PW)], idx_v)
    off = (base // S) * HW
    for j in range(PW // 16):
        gidx_v[pl.ds(j * 16, 16)] = idx_v[pl.ds(j * 16, 16)] + off
    pltpu.async_copy(bnd_ref.at[gidx_v], out_v, sem).wait()
    pltpu.sync_copy(out_v, out_ref.at[pl.ds(base, PW)])


def _loss_body(sr_ref, vr_ref, br_ref, sc_ref, vc_ref, bc_ref, out_ref, *, k):
    x_row = br_ref[...] * -jnp.log(_EPS + vr_ref[...] / sr_ref[...])
    x_col = bc_ref[...] * -jnp.log(_EPS + vc_ref[...] / sc_ref[...])
    le = (x_row <= x_col).astype(jnp.float32)
    cnt = jnp.sum(le, axis=1, keepdims=True)
    elig = cnt >= jnp.float32(k + 1)
    thr = jnp.min(jnp.where(elig, x_col, jnp.inf), keepdims=True)
    m = x_row > thr
    numer = jnp.sum(jnp.where(m, x_row, 0.0), axis=1, keepdims=True)
    denom = 1.0 + jnp.sum(jnp.where(m, br_ref[...], 0.0), axis=1, keepdims=True)
    out_ref[...] = numer / denom


def kernel(response_map, source_feature_1d_locations, boundaries):
    B, S, H, W = response_map.shape
    HW = H * W
    N = B * S
    R = _R
    NSB = S // R
    PW = N // _NW

    idx_i32 = source_feature_1d_locations.astype(jnp.int32)

    NBUF = 4
    R3 = 64
    GROUP = NBUF * R3
    NG = N // GROUP
    idx3 = idx_i32.reshape(NG, 1, GROUP)

    sums, vals = pl.pallas_call(
        functools.partial(_rows_dma_body, W=W, S=S, R3=R3, NBUF=NBUF, NG=NG),
        grid=(NG,),
        in_specs=[
            pl.BlockSpec((1, 1, GROUP), lambda g: (g, 0, 0)),
            pl.BlockSpec(memory_space=pl.ANY),
        ],
        out_specs=[
            pl.BlockSpec((1, 1, GROUP), lambda g: (g, 0, 0)),
            pl.BlockSpec((1, 1, GROUP), lambda g: (g, 0, 0)),
        ],
        out_shape=[
            jax.ShapeDtypeStruct((NG, 1, GROUP), jnp.float32),
            jax.ShapeDtypeStruct((NG, 1, GROUP), jnp.float32),
        ],
        scratch_shapes=(
            [pltpu.VMEM((R3, H, W), jnp.float32) for _ in range(NBUF)]
            + [pltpu.SemaphoreType.DMA for _ in range(NBUF)]),
        compiler_params=pltpu.CompilerParams(
            vmem_limit_bytes=100 * 1024 * 1024),
    )(idx3, response_map.reshape(N, H, W))

    sb_flat = pl.kernel(
        functools.partial(_sc_gather_body, S=S, HW=HW, PW=PW),
        out_type=jax.ShapeDtypeStruct((N,), jnp.float32),
        mesh=plsc.VectorSubcoreMesh(core_axis_name="c", subcore_axis_name="s",
                                    num_cores=_NC, num_subcores=_NS),
        scratch_types=[
            pltpu.VMEM((PW,), jnp.int32),
            pltpu.VMEM((PW,), jnp.int32),
            pltpu.VMEM((PW,), jnp.float32),
            pltpu.SemaphoreType.DMA,
        ],
    )(boundaries.reshape(B * HW), idx_i32.reshape(N))

    k = int(N * _CUTPER)
    sr = sums.reshape(1, N)
    vr = vals.reshape(1, N)
    br = sb_flat.reshape(1, N)

    loss = pl.pallas_call(
        functools.partial(_loss_body, k=k),
        out_shape=jax.ShapeDtypeStruct((1, 1), jnp.float32),
    )(sr, vr, br, sr.reshape(N, 1), vr.reshape(N, 1), br.reshape(N, 1))

    return loss.reshape(())
